# Initial kernel scaffold; baseline (speedup 1.0000x reference)
import jax
import jax.numpy as jnp
from jax.experimental import pallas as pl


def kernel(x, edge_index, W, att_src, att_dst):
    raise NotImplementedError("write your pallas kernel here")


# speedup vs baseline: 8.7985x; 8.7985x over previous
import functools

import jax
import jax.numpy as jnp
from jax import lax
from jax.experimental import pallas as pl
from jax.experimental.pallas import tpu as pltpu
from jax.experimental.pallas import tpu_sc as plsc
from jax._src.pallas import mpmd as _mpmd

N = 10000
E = 320000
D = 128

NC = 2
NS = 16
NW = NC * NS
L = 16

NPAD = 10240
EPT = E // NW
RED = NPAD // NS
K = 128
NCH = EPT // K
TAIL = EPT - NCH * K

BN = 512


def _tc1_body(x_ref, w_ref, a8_ref, h_ref, at_ref):
    h = lax.dot_general(x_ref[...], w_ref[...], (((1,), (1,)), ((), ())),
                        preferred_element_type=jnp.float32)
    h_ref[...] = h
    at_ref[...] = lax.dot_general(a8_ref[...], h, (((1,), (1,)), ((), ())),
                                  preferred_element_type=jnp.float32)


_tc1 = pl.pallas_call(
    _tc1_body,
    grid=(pl.cdiv(N, BN),),
    in_specs=[
        pl.BlockSpec((BN, D), lambda i: (i, 0)),
        pl.BlockSpec((D, D), lambda i: (0, 0)),
        pl.BlockSpec((8, D), lambda i: (0, 0)),
    ],
    out_specs=[
        pl.BlockSpec((BN, D), lambda i: (i, 0)),
        pl.BlockSpec((8, BN), lambda i: (0, i)),
    ],
    out_shape=[
        jax.ShapeDtypeStruct((N, D), jnp.float32),
        jax.ShapeDtypeStruct((8, N), jnp.float32),
    ],
)


def _sc1_body(src_hbm, dst_hbm, as_hbm, ad_hbm,
              ex_hbm, dena_hbm, denb_hbm,
              as_v, ad_v, src_v, dst_v, ex_v, den_loc, red_v, tmp_v,
              den_stage):
    c = lax.axis_index("c")
    s = lax.axis_index("s")
    wid = c * NS + s
    base = wid * EPT

    pltpu.sync_copy(as_hbm, as_v)
    pltpu.sync_copy(ad_hbm, ad_v)
    pltpu.sync_copy(src_hbm.at[pl.ds(base, EPT)], src_v)
    pltpu.sync_copy(dst_hbm.at[pl.ds(base, EPT)], dst_v)

    def zero_den(i, _):
        den_loc[pl.ds(i * L, L)] = jnp.zeros((L,), jnp.float32)
        return 0
    lax.fori_loop(0, NPAD // L, zero_den, 0)

    def edge_grp(i, _):
        off = i * L
        s16 = src_v[pl.ds(off, L)]
        d16 = dst_v[pl.ds(off, L)]
        a = plsc.load_gather(as_v, [s16])
        b = plsc.load_gather(ad_v, [d16])
        e = a + b
        e = jnp.where(e > 0, e, e * jnp.float32(0.2))
        ex = jnp.exp(e)
        ex_v[pl.ds(off, L)] = ex
        plsc.addupdate_scatter(den_loc, [d16], ex)
        return 0
    lax.fori_loop(0, EPT // L, edge_grp, 0)

    pltpu.sync_copy(ex_v, ex_hbm.at[pl.ds(base, EPT)])

    pltpu.sync_copy(den_loc, den_stage.at[s])
    plsc.subcore_barrier()

    def zero_red(i, _):
        red_v[pl.ds(i * L, L)] = jnp.zeros((L,), jnp.float32)
        return 0
    lax.fori_loop(0, RED // L, zero_red, 0)

    for k in range(NS):
        pltpu.sync_copy(den_stage.at[k, pl.ds(s * RED, RED)], tmp_v)

        def acc(i, _):
            red_v[pl.ds(i * L, L)] = (red_v[pl.ds(i * L, L)]
                                      + tmp_v[pl.ds(i * L, L)])
            return 0
        lax.fori_loop(0, RED // L, acc, 0)

    @pl.when(c == 0)
    def _():
        pltpu.sync_copy(red_v, dena_hbm.at[pl.ds(s * RED, RED)])

    @pl.when(c == 1)
    def _():
        pltpu.sync_copy(red_v, denb_hbm.at[pl.ds(s * RED, RED)])


_sc1 = pl.kernel(
    _sc1_body,
    out_type=[
        jax.ShapeDtypeStruct((E,), jnp.float32),
        jax.ShapeDtypeStruct((NPAD,), jnp.float32),
        jax.ShapeDtypeStruct((NPAD,), jnp.float32),
    ],
    mesh=plsc.VectorSubcoreMesh(core_axis_name="c", subcore_axis_name="s"),
    compiler_params=pltpu.CompilerParams(needs_layout_passes=False),
    scratch_types=[
        pltpu.VMEM((N,), jnp.float32),
        pltpu.VMEM((N,), jnp.float32),
        pltpu.VMEM((EPT,), jnp.int32),
        pltpu.VMEM((EPT,), jnp.int32),
        pltpu.VMEM((EPT,), jnp.float32),
        pltpu.VMEM((NPAD,), jnp.float32),
        pltpu.VMEM((RED,), jnp.float32),
        pltpu.VMEM((RED,), jnp.float32),
        pltpu.MemorySpace.VMEM_SHARED((NS, NPAD), jnp.float32),
    ],
)


DH = D // 2


def _sc2_body(src_hbm, dst_hbm, ex_hbm, h2_hbm, dena_hbm, denb_hbm, zer_hbm,
              attn_hbm, agga0_hbm, agga1_hbm, aggb0_hbm, aggb1_hbm, denf_hbm,
              den_v, d2_v, src_v, dst_v, ex_v, alpha_full,
              sidx_v, didx_v, fidx_v, alpha_v, rows_v, zrow_v,
              agg_sh):
    del zer_hbm
    c = lax.axis_index("c")
    s = lax.axis_index("s")
    wid = c * NS + s
    base = wid * EPT

    pltpu.sync_copy(dena_hbm, den_v)
    pltpu.sync_copy(denb_hbm, d2_v)

    def dsum(i, _):
        den_v[pl.ds(i * L, L)] = den_v[pl.ds(i * L, L)] + d2_v[pl.ds(i * L, L)]
        return 0
    lax.fori_loop(0, NPAD // L, dsum, 0)

    @pl.when(c == 0)
    def _():
        pltpu.sync_copy(den_v.at[pl.ds(s * RED, RED)],
                        denf_hbm.at[pl.ds(s * RED, RED)])

    def zero_zrow(q, _):
        i = q // (DH // L)
        p = q % (DH // L)
        zrow_v[i, pl.ds(p * L, L)] = jnp.zeros((L,), jnp.float32)
        return 0
    lax.fori_loop(0, K * (DH // L), zero_zrow, 0)

    pltpu.sync_copy(src_hbm.at[pl.ds(base, EPT)], src_v)
    pltpu.sync_copy(dst_hbm.at[pl.ds(base, EPT)], dst_v)
    pltpu.sync_copy(ex_hbm.at[pl.ds(base, EPT)], ex_v)

    agg_outs = ((agga0_hbm, agga1_hbm), (aggb0_hbm, aggb1_hbm))

    for f in range(2):
        for i in range(RED // K):
            pltpu.sync_copy(zrow_v, agg_sh.at[pl.ds(s * RED + i * K, K)])
        plsc.subcore_barrier()

        def chunk(ci, _):
            cbase = ci * K

            def grp(g, _):
                off = cbase + g * L
                s16 = src_v[pl.ds(off, L)]
                d16 = dst_v[pl.ds(off, L)]
                o = g * L
                if f == 0:
                    e16 = ex_v[pl.ds(off, L)]
                    dn = plsc.load_gather(den_v, [d16])
                    al = e16 / dn
                    alpha_full[pl.ds(off, L)] = al
                    fidx_v[pl.ds(o, L)] = s16 * jnp.int32(N) + d16
                else:
                    al = alpha_full[pl.ds(off, L)]
                alpha_v[pl.ds(o, L)] = al
                sidx_v[pl.ds(o, L)] = s16 * jnp.int32(2) + jnp.int32(f)
                didx_v[pl.ds(o, L)] = d16
                return 0
            lax.fori_loop(0, K // L, grp, 0)

            pltpu.sync_copy(h2_hbm.at[sidx_v], rows_v)

            def scale(j, _):
                av = plsc.load_gather(alpha_v, [jnp.full((L,), j, jnp.int32)])
                for p in range(DH // L):
                    r = rows_v[j, pl.ds(p * L, L)]
                    rows_v[j, pl.ds(p * L, L)] = r * av
                return 0
            lax.fori_loop(0, K, scale, 0)

            pltpu.sync_copy(rows_v, agg_sh.at[didx_v], add=True)
            if f == 0:
                pltpu.sync_copy(alpha_v, attn_hbm.at[fidx_v])
            return 0
        lax.fori_loop(0, NCH, chunk, 0)

        toff = NCH * K
        s16 = src_v[pl.ds(toff, L)]
        d16 = dst_v[pl.ds(toff, L)]
        if f == 0:
            e16 = ex_v[pl.ds(toff, L)]
            dn = plsc.load_gather(den_v, [d16])
            al = e16 / dn
            alpha_full[pl.ds(toff, L)] = al
            fidx_v[pl.ds(0, L)] = s16 * jnp.int32(N) + d16
        else:
            al = alpha_full[pl.ds(toff, L)]
        alpha_v[pl.ds(0, L)] = al
        pltpu.sync_copy(h2_hbm.at[s16 * jnp.int32(2) + jnp.int32(f)],
                        rows_v.at[pl.ds(0, L)])

        def tscale(j, _):
            av = plsc.load_gather(alpha_v, [jnp.full((L,), j, jnp.int32)])
            for p in range(DH // L):
                r = rows_v[j, pl.ds(p * L, L)]
                rows_v[j, pl.ds(p * L, L)] = r * av
            return 0
        lax.fori_loop(0, L, tscale, 0)

        pltpu.sync_copy(rows_v.at[pl.ds(0, L)], agg_sh.at[d16], add=True)
        if f == 0:
            f16 = fidx_v[pl.ds(0, L)]
            pltpu.sync_copy(alpha_v.at[pl.ds(0, L)], attn_hbm.at[f16])

        plsc.subcore_barrier()

        @pl.when(c == 0)
        def _():
            pltpu.sync_copy(agg_sh.at[pl.ds(s * RED, RED)],
                            agg_outs[0][f].at[pl.ds(s * RED, RED)])

        @pl.when(c == 1)
        def _():
            pltpu.sync_copy(agg_sh.at[pl.ds(s * RED, RED)],
                            agg_outs[1][f].at[pl.ds(s * RED, RED)])

        plsc.subcore_barrier()


_sc2 = _mpmd._mpmd_map(
    [(plsc.VectorSubcoreMesh(core_axis_name="c", subcore_axis_name="s"),
      _sc2_body)],
    [
        jax.ShapeDtypeStruct((N * N,), jnp.float32),
        jax.ShapeDtypeStruct((NPAD, DH), jnp.float32),
        jax.ShapeDtypeStruct((NPAD, DH), jnp.float32),
        jax.ShapeDtypeStruct((NPAD, DH), jnp.float32),
        jax.ShapeDtypeStruct((NPAD, DH), jnp.float32),
        jax.ShapeDtypeStruct((NPAD,), jnp.float32),
    ],
    input_output_aliases={6: 0},
    compiler_params=pltpu.CompilerParams(needs_layout_passes=False,
                                         use_tc_tiling_on_sc=False),
    scratch_types=[
        pltpu.VMEM((NPAD,), jnp.float32),
        pltpu.VMEM((NPAD,), jnp.float32),
        pltpu.VMEM((EPT,), jnp.int32),
        pltpu.VMEM((EPT,), jnp.int32),
        pltpu.VMEM((EPT,), jnp.float32),
        pltpu.VMEM((EPT,), jnp.float32),
        pltpu.VMEM((K,), jnp.int32),
        pltpu.VMEM((K,), jnp.int32),
        pltpu.VMEM((K,), jnp.int32),
        pltpu.VMEM((K,), jnp.float32),
        pltpu.VMEM((K, DH), jnp.float32),
        pltpu.VMEM((K, DH), jnp.float32),
        pltpu.MemorySpace.VMEM_SHARED((NPAD, DH), jnp.float32),
    ],
)


def _tc2_body(a00_ref, a01_ref, b00_ref, b01_ref, den_ref, h_ref, o_ref):
    lo = a00_ref[...] + b00_ref[...]
    hi = a01_ref[...] + b01_ref[...]
    ssum = jnp.concatenate([lo, hi], axis=1)
    den = den_ref[...]
    o_ref[...] = jnp.where(den > 0, ssum, h_ref[...])


_tc2 = pl.pallas_call(
    _tc2_body,
    grid=(pl.cdiv(N, BN),),
    in_specs=[
        pl.BlockSpec((BN, DH), lambda i: (i, 0)),
        pl.BlockSpec((BN, DH), lambda i: (i, 0)),
        pl.BlockSpec((BN, DH), lambda i: (i, 0)),
        pl.BlockSpec((BN, DH), lambda i: (i, 0)),
        pl.BlockSpec((BN, 1), lambda i: (i, 0)),
        pl.BlockSpec((BN, D), lambda i: (i, 0)),
    ],
    out_specs=pl.BlockSpec((BN, D), lambda i: (i, 0)),
    out_shape=jax.ShapeDtypeStruct((N, D), jnp.float32),
)


def kernel(x, edge_index, W, att_src, att_dst):
    a8 = jnp.zeros((8, D), jnp.float32).at[0].set(att_src[0]).at[1].set(att_dst[0])
    h, asadT = _tc1(x, W, a8)
    src = edge_index[0]
    dst = edge_index[1]
    as_arr = asadT[0]
    ad_arr = asadT[1]
    ex, den_a, den_b = _sc1(src, dst, as_arr, ad_arr)
    zer = jnp.zeros((N * N,), jnp.float32)
    h2 = jnp.reshape(h, (2 * N, DH))
    attnflat, agg_a0, agg_a1, agg_b0, agg_b1, denf = _sc2(
        src, dst, ex, h2, den_a, den_b, zer)
    out = _tc2(agg_a0, agg_a1, agg_b0, agg_b1,
               jnp.reshape(denf[:N], (N, 1)), h)
    return out, attnflat.reshape(N, N)


# speedup vs baseline: 9.4978x; 1.0795x over previous
import functools

import jax
import jax.numpy as jnp
from jax import lax
from jax.experimental import pallas as pl
from jax.experimental.pallas import tpu as pltpu
from jax.experimental.pallas import tpu_sc as plsc
from jax._src.pallas import mpmd as _mpmd

N = 10000
E = 320000
D = 128

NC = 2
NS = 16
NW = NC * NS
L = 16

NPAD = 10240
EPT = E // NW
RED = NPAD // NS
K = 128
NCH = EPT // K
TAIL = EPT - NCH * K

BN = 512


def _tc1_body(x_ref, w_ref, a8_ref, h_ref, at_ref):
    h = lax.dot_general(x_ref[...], w_ref[...], (((1,), (1,)), ((), ())),
                        preferred_element_type=jnp.float32)
    h_ref[...] = h
    at_ref[...] = lax.dot_general(a8_ref[...], h, (((1,), (1,)), ((), ())),
                                  preferred_element_type=jnp.float32)


_tc1 = pl.pallas_call(
    _tc1_body,
    grid=(pl.cdiv(N, BN),),
    in_specs=[
        pl.BlockSpec((BN, D), lambda i: (i, 0)),
        pl.BlockSpec((D, D), lambda i: (0, 0)),
        pl.BlockSpec((8, D), lambda i: (0, 0)),
    ],
    out_specs=[
        pl.BlockSpec((BN, D), lambda i: (i, 0)),
        pl.BlockSpec((8, BN), lambda i: (0, i)),
    ],
    out_shape=[
        jax.ShapeDtypeStruct((N, D), jnp.float32),
        jax.ShapeDtypeStruct((8, N), jnp.float32),
    ],
)


def _sc1_body(src_hbm, dst_hbm, as_hbm, ad_hbm,
              ex_hbm, dena_hbm, denb_hbm,
              as_v, ad_v, src_v, dst_v, ex_v, den_loc, red_v, tmp_v,
              den_stage):
    c = lax.axis_index("c")
    s = lax.axis_index("s")
    wid = c * NS + s
    base = wid * EPT

    pltpu.sync_copy(as_hbm, as_v)
    pltpu.sync_copy(ad_hbm, ad_v)
    pltpu.sync_copy(src_hbm.at[pl.ds(base, EPT)], src_v)
    pltpu.sync_copy(dst_hbm.at[pl.ds(base, EPT)], dst_v)

    def zero_den(i, _):
        den_loc[pl.ds(i * L, L)] = jnp.zeros((L,), jnp.float32)
        return 0
    lax.fori_loop(0, NPAD // L, zero_den, 0)

    def edge_grp(i, _):
        off = i * L
        s16 = src_v[pl.ds(off, L)]
        d16 = dst_v[pl.ds(off, L)]
        a = plsc.load_gather(as_v, [s16])
        b = plsc.load_gather(ad_v, [d16])
        e = a + b
        e = jnp.where(e > 0, e, e * jnp.float32(0.2))
        ex = jnp.exp(e)
        ex_v[pl.ds(off, L)] = ex
        plsc.addupdate_scatter(den_loc, [d16], ex)
        return 0
    lax.fori_loop(0, EPT // L, edge_grp, 0)

    pltpu.sync_copy(ex_v, ex_hbm.at[pl.ds(base, EPT)])

    pltpu.sync_copy(den_loc, den_stage.at[s])
    plsc.subcore_barrier()

    def zero_red(i, _):
        red_v[pl.ds(i * L, L)] = jnp.zeros((L,), jnp.float32)
        return 0
    lax.fori_loop(0, RED // L, zero_red, 0)

    for k in range(NS):
        pltpu.sync_copy(den_stage.at[k, pl.ds(s * RED, RED)], tmp_v)

        def acc(i, _):
            red_v[pl.ds(i * L, L)] = (red_v[pl.ds(i * L, L)]
                                      + tmp_v[pl.ds(i * L, L)])
            return 0
        lax.fori_loop(0, RED // L, acc, 0)

    @pl.when(c == 0)
    def _():
        pltpu.sync_copy(red_v, dena_hbm.at[pl.ds(s * RED, RED)])

    @pl.when(c == 1)
    def _():
        pltpu.sync_copy(red_v, denb_hbm.at[pl.ds(s * RED, RED)])


_sc1 = pl.kernel(
    _sc1_body,
    out_type=[
        jax.ShapeDtypeStruct((E,), jnp.float32),
        jax.ShapeDtypeStruct((NPAD,), jnp.float32),
        jax.ShapeDtypeStruct((NPAD,), jnp.float32),
    ],
    mesh=plsc.VectorSubcoreMesh(core_axis_name="c", subcore_axis_name="s"),
    compiler_params=pltpu.CompilerParams(needs_layout_passes=False),
    scratch_types=[
        pltpu.VMEM((N,), jnp.float32),
        pltpu.VMEM((N,), jnp.float32),
        pltpu.VMEM((EPT,), jnp.int32),
        pltpu.VMEM((EPT,), jnp.int32),
        pltpu.VMEM((EPT,), jnp.float32),
        pltpu.VMEM((NPAD,), jnp.float32),
        pltpu.VMEM((RED,), jnp.float32),
        pltpu.VMEM((RED,), jnp.float32),
        pltpu.MemorySpace.VMEM_SHARED((NS, NPAD), jnp.float32),
    ],
)


DH = D // 2


def _sc2_body(src_hbm, dst_hbm, ex_hbm, h2_hbm, dena_hbm, denb_hbm, zer_hbm,
              attn_hbm, agga0_hbm, agga1_hbm, aggb0_hbm, aggb1_hbm, denf_hbm,
              den_v, d2_v, src_v, dst_v, ex_v, alpha_full,
              sidx_v, didx_v, fidx_v, alpha_v, rows_v, zrow_v,
              gsem, agg_sh):
    del zer_hbm
    c = lax.axis_index("c")
    s = lax.axis_index("s")
    wid = c * NS + s
    base = wid * EPT

    pltpu.sync_copy(dena_hbm, den_v)
    pltpu.sync_copy(denb_hbm, d2_v)

    def dsum(i, _):
        den_v[pl.ds(i * L, L)] = den_v[pl.ds(i * L, L)] + d2_v[pl.ds(i * L, L)]
        return 0
    lax.fori_loop(0, NPAD // L, dsum, 0)

    @pl.when(c == 0)
    def _():
        pltpu.sync_copy(den_v.at[pl.ds(s * RED, RED)],
                        denf_hbm.at[pl.ds(s * RED, RED)])

    def zero_zrow(q, _):
        i = q // (DH // L)
        p = q % (DH // L)
        zrow_v[i, pl.ds(p * L, L)] = jnp.zeros((L,), jnp.float32)
        return 0
    lax.fori_loop(0, K * (DH // L), zero_zrow, 0)

    pltpu.sync_copy(src_hbm.at[pl.ds(base, EPT)], src_v)
    pltpu.sync_copy(dst_hbm.at[pl.ds(base, EPT)], dst_v)
    pltpu.sync_copy(ex_hbm.at[pl.ds(base, EPT)], ex_v)

    agg_outs = ((agga0_hbm, agga1_hbm), (aggb0_hbm, aggb1_hbm))

    for f in range(2):
        for i in range(RED // K):
            pltpu.sync_copy(zrow_v, agg_sh.at[pl.ds(s * RED + i * K, K)])
        plsc.subcore_barrier()

        def grp_fill(ci, slot):
            cbase = ci * K

            def grp(g, _):
                off = cbase + g * L
                s16 = src_v[pl.ds(off, L)]
                d16 = dst_v[pl.ds(off, L)]
                o = g * L
                if f == 0:
                    e16 = ex_v[pl.ds(off, L)]
                    dn = plsc.load_gather(den_v, [d16])
                    al = e16 / dn
                    alpha_full[pl.ds(off, L)] = al
                    fidx_v[slot, pl.ds(o, L)] = s16 * jnp.int32(N) + d16
                else:
                    al = alpha_full[pl.ds(off, L)]
                alpha_v[slot, pl.ds(o, L)] = al
                sidx_v[slot, pl.ds(o, L)] = s16 * jnp.int32(2) + jnp.int32(f)
                didx_v[slot, pl.ds(o, L)] = d16
                return 0
            lax.fori_loop(0, K // L, grp, 0)

        def start_gather(ci, slot):
            grp_fill(ci, slot)
            pltpu.async_copy(h2_hbm.at[sidx_v.at[slot]], rows_v.at[slot],
                             gsem)

        start_gather(jnp.int32(0), jnp.int32(0))

        def chunk(ci, _):
            slot = lax.rem(ci, 2)
            pltpu.make_async_copy(h2_hbm.at[sidx_v.at[slot]],
                                  rows_v.at[slot], gsem).wait()

            @pl.when(ci + 1 < NCH)
            def _():
                start_gather(ci + 1, lax.rem(ci + 1, 2))

            def scale(j, _):
                av = plsc.load_gather(
                    alpha_v, [jnp.full((L,), slot, jnp.int32),
                              jnp.full((L,), j, jnp.int32)])
                for p in range(DH // L):
                    r = rows_v[slot, j, pl.ds(p * L, L)]
                    rows_v[slot, j, pl.ds(p * L, L)] = r * av
                return 0
            lax.fori_loop(0, K, scale, 0)

            pltpu.sync_copy(rows_v.at[slot], agg_sh.at[didx_v.at[slot]],
                            add=True)
            if f == 0:
                pltpu.sync_copy(alpha_v.at[slot], attn_hbm.at[fidx_v.at[slot]])
            return 0
        lax.fori_loop(0, NCH, chunk, 0)

        toff = NCH * K
        s16 = src_v[pl.ds(toff, L)]
        d16 = dst_v[pl.ds(toff, L)]
        if f == 0:
            e16 = ex_v[pl.ds(toff, L)]
            dn = plsc.load_gather(den_v, [d16])
            al = e16 / dn
            alpha_full[pl.ds(toff, L)] = al
            fidx_v[0, pl.ds(0, L)] = s16 * jnp.int32(N) + d16
        else:
            al = alpha_full[pl.ds(toff, L)]
        alpha_v[0, pl.ds(0, L)] = al
        pltpu.sync_copy(h2_hbm.at[s16 * jnp.int32(2) + jnp.int32(f)],
                        rows_v.at[0, pl.ds(0, L)])

        def tscale(j, _):
            av = plsc.load_gather(
                alpha_v, [jnp.zeros((L,), jnp.int32),
                          jnp.full((L,), j, jnp.int32)])
            for p in range(DH // L):
                r = rows_v[0, j, pl.ds(p * L, L)]
                rows_v[0, j, pl.ds(p * L, L)] = r * av
            return 0
        lax.fori_loop(0, L, tscale, 0)

        pltpu.sync_copy(rows_v.at[0, pl.ds(0, L)], agg_sh.at[d16], add=True)
        if f == 0:
            f16 = fidx_v[0, pl.ds(0, L)]
            pltpu.sync_copy(alpha_v.at[0, pl.ds(0, L)], attn_hbm.at[f16])

        plsc.subcore_barrier()

        @pl.when(c == 0)
        def _():
            pltpu.sync_copy(agg_sh.at[pl.ds(s * RED, RED)],
                            agg_outs[0][f].at[pl.ds(s * RED, RED)])

        @pl.when(c == 1)
        def _():
            pltpu.sync_copy(agg_sh.at[pl.ds(s * RED, RED)],
                            agg_outs[1][f].at[pl.ds(s * RED, RED)])

        plsc.subcore_barrier()


_sc2 = _mpmd._mpmd_map(
    [(plsc.VectorSubcoreMesh(core_axis_name="c", subcore_axis_name="s"),
      _sc2_body)],
    [
        jax.ShapeDtypeStruct((N * N,), jnp.float32),
        jax.ShapeDtypeStruct((NPAD, DH), jnp.float32),
        jax.ShapeDtypeStruct((NPAD, DH), jnp.float32),
        jax.ShapeDtypeStruct((NPAD, DH), jnp.float32),
        jax.ShapeDtypeStruct((NPAD, DH), jnp.float32),
        jax.ShapeDtypeStruct((NPAD,), jnp.float32),
    ],
    input_output_aliases={6: 0},
    compiler_params=pltpu.CompilerParams(needs_layout_passes=False,
                                         use_tc_tiling_on_sc=False),
    scratch_types=[
        pltpu.VMEM((NPAD,), jnp.float32),
        pltpu.VMEM((NPAD,), jnp.float32),
        pltpu.VMEM((EPT,), jnp.int32),
        pltpu.VMEM((EPT,), jnp.int32),
        pltpu.VMEM((EPT,), jnp.float32),
        pltpu.VMEM((EPT,), jnp.float32),
        pltpu.VMEM((2, K), jnp.int32),
        pltpu.VMEM((2, K), jnp.int32),
        pltpu.VMEM((2, K), jnp.int32),
        pltpu.VMEM((2, K), jnp.float32),
        pltpu.VMEM((2, K, DH), jnp.float32),
        pltpu.VMEM((K, DH), jnp.float32),
        pltpu.SemaphoreType.DMA,
        pltpu.MemorySpace.VMEM_SHARED((NPAD, DH), jnp.float32),
    ],
)


def _tc2_body(a00_ref, a01_ref, b00_ref, b01_ref, den_ref, h_ref, o_ref):
    lo = a00_ref[...] + b00_ref[...]
    hi = a01_ref[...] + b01_ref[...]
    ssum = jnp.concatenate([lo, hi], axis=1)
    den = den_ref[...]
    o_ref[...] = jnp.where(den > 0, ssum, h_ref[...])


_tc2 = pl.pallas_call(
    _tc2_body,
    grid=(pl.cdiv(N, BN),),
    in_specs=[
        pl.BlockSpec((BN, DH), lambda i: (i, 0)),
        pl.BlockSpec((BN, DH), lambda i: (i, 0)),
        pl.BlockSpec((BN, DH), lambda i: (i, 0)),
        pl.BlockSpec((BN, DH), lambda i: (i, 0)),
        pl.BlockSpec((BN, 1), lambda i: (i, 0)),
        pl.BlockSpec((BN, D), lambda i: (i, 0)),
    ],
    out_specs=pl.BlockSpec((BN, D), lambda i: (i, 0)),
    out_shape=jax.ShapeDtypeStruct((N, D), jnp.float32),
)


def kernel(x, edge_index, W, att_src, att_dst):
    a8 = jnp.zeros((8, D), jnp.float32).at[0].set(att_src[0]).at[1].set(att_dst[0])
    h, asadT = _tc1(x, W, a8)
    src = edge_index[0]
    dst = edge_index[1]
    as_arr = asadT[0]
    ad_arr = asadT[1]
    ex, den_a, den_b = _sc1(src, dst, as_arr, ad_arr)
    zer = jnp.zeros((N * N,), jnp.float32)
    h2 = jnp.reshape(h, (2 * N, DH))
    attnflat, agg_a0, agg_a1, agg_b0, agg_b1, denf = _sc2(
        src, dst, ex, h2, den_a, den_b, zer)
    out = _tc2(agg_a0, agg_a1, agg_b0, agg_b1,
               jnp.reshape(denf[:N], (N, 1)), h)
    return out, attnflat.reshape(N, N)


# speedup vs baseline: 10.0679x; 1.0600x over previous
import functools

import jax
import jax.numpy as jnp
from jax import lax
from jax.experimental import pallas as pl
from jax.experimental.pallas import tpu as pltpu
from jax.experimental.pallas import tpu_sc as plsc
from jax._src.pallas import mpmd as _mpmd

N = 10000
E = 320000
D = 128

NC = 2
NS = 16
NW = NC * NS
L = 16

NPAD = 10240
EPT = E // NW
RED = NPAD // NS
K = 128
NCH = EPT // K
TAIL = EPT - NCH * K

BN = 512
DH = D // 2


def _tc1_body(x_ref, w_ref, a8_ref, h_ref, at_ref):
    h = lax.dot_general(x_ref[...], w_ref[...], (((1,), (1,)), ((), ())),
                        preferred_element_type=jnp.float32)
    h_ref[...] = h
    at_ref[...] = lax.dot_general(a8_ref[...], h, (((1,), (1,)), ((), ())),
                                  preferred_element_type=jnp.float32)


_tc1 = pl.pallas_call(
    _tc1_body,
    grid=(pl.cdiv(N, BN),),
    in_specs=[
        pl.BlockSpec((BN, D), lambda i: (i, 0)),
        pl.BlockSpec((D, D), lambda i: (0, 0)),
        pl.BlockSpec((8, D), lambda i: (0, 0)),
    ],
    out_specs=[
        pl.BlockSpec((BN, D), lambda i: (i, 0)),
        pl.BlockSpec((8, BN), lambda i: (0, i)),
    ],
    out_shape=[
        jax.ShapeDtypeStruct((N, D), jnp.float32),
        jax.ShapeDtypeStruct((8, N), jnp.float32),
    ],
)


def _sc1_body(src_hbm, dst_hbm, as_hbm, ad_hbm,
              ex_hbm, dena_hbm, denb_hbm,
              as_v, ad_v, src_v, dst_v, ex_v, den_loc, red_v, tmp_v,
              den_stage):
    c = lax.axis_index("c")
    s = lax.axis_index("s")
    wid = c * NS + s
    base = wid * EPT

    pltpu.sync_copy(as_hbm, as_v)
    pltpu.sync_copy(ad_hbm, ad_v)
    pltpu.sync_copy(src_hbm.at[pl.ds(base, EPT)], src_v)
    pltpu.sync_copy(dst_hbm.at[pl.ds(base, EPT)], dst_v)

    def zero_den(i, _):
        den_loc[pl.ds(i * L, L)] = jnp.zeros((L,), jnp.float32)
        return 0
    lax.fori_loop(0, NPAD // L, zero_den, 0)

    def edge_grp(i, _):
        off = i * L
        s16 = src_v[pl.ds(off, L)]
        d16 = dst_v[pl.ds(off, L)]
        a = plsc.load_gather(as_v, [s16])
        b = plsc.load_gather(ad_v, [d16])
        e = a + b
        e = jnp.where(e > 0, e, e * jnp.float32(0.2))
        ex = jnp.exp(e)
        ex_v[pl.ds(off, L)] = ex
        plsc.addupdate_scatter(den_loc, [d16], ex)
        return 0
    lax.fori_loop(0, EPT // L, edge_grp, 0)

    pltpu.sync_copy(ex_v, ex_hbm.at[pl.ds(base, EPT)])

    pltpu.sync_copy(den_loc, den_stage.at[s])
    plsc.subcore_barrier()

    def zero_red(i, _):
        red_v[pl.ds(i * L, L)] = jnp.zeros((L,), jnp.float32)
        return 0
    lax.fori_loop(0, RED // L, zero_red, 0)

    for k in range(NS):
        pltpu.sync_copy(den_stage.at[k, pl.ds(s * RED, RED)], tmp_v)

        def acc(i, _):
            red_v[pl.ds(i * L, L)] = (red_v[pl.ds(i * L, L)]
                                      + tmp_v[pl.ds(i * L, L)])
            return 0
        lax.fori_loop(0, RED // L, acc, 0)

    @pl.when(c == 0)
    def _():
        pltpu.sync_copy(red_v, dena_hbm.at[pl.ds(s * RED, RED)])

    @pl.when(c == 1)
    def _():
        pltpu.sync_copy(red_v, denb_hbm.at[pl.ds(s * RED, RED)])


_sc1 = pl.kernel(
    _sc1_body,
    out_type=[
        jax.ShapeDtypeStruct((E,), jnp.float32),
        jax.ShapeDtypeStruct((NPAD,), jnp.float32),
        jax.ShapeDtypeStruct((NPAD,), jnp.float32),
    ],
    mesh=plsc.VectorSubcoreMesh(core_axis_name="c", subcore_axis_name="s"),
    compiler_params=pltpu.CompilerParams(needs_layout_passes=False),
    scratch_types=[
        pltpu.VMEM((N,), jnp.float32),
        pltpu.VMEM((N,), jnp.float32),
        pltpu.VMEM((EPT,), jnp.int32),
        pltpu.VMEM((EPT,), jnp.int32),
        pltpu.VMEM((EPT,), jnp.float32),
        pltpu.VMEM((NPAD,), jnp.float32),
        pltpu.VMEM((RED,), jnp.float32),
        pltpu.VMEM((RED,), jnp.float32),
        pltpu.MemorySpace.VMEM_SHARED((NS, NPAD), jnp.float32),
    ],
)


def _sc2a_body(src_hbm, dst_hbm, ex_hbm, dena_hbm, denb_hbm, zer_hbm,
               attn_hbm, alpha_hbm, denf_hbm,
               den_v, d2_v, src_v, dst_v, ex_v, alpha_full, fidx2_v,
               asem):
    del zer_hbm
    c = lax.axis_index("c")
    s = lax.axis_index("s")
    wid = c * NS + s
    base = wid * EPT

    pltpu.sync_copy(dena_hbm, den_v)
    pltpu.sync_copy(denb_hbm, d2_v)

    def dsum(i, _):
        den_v[pl.ds(i * L, L)] = den_v[pl.ds(i * L, L)] + d2_v[pl.ds(i * L, L)]
        return 0
    lax.fori_loop(0, NPAD // L, dsum, 0)

    @pl.when(c == 0)
    def _():
        pltpu.sync_copy(den_v.at[pl.ds(s * RED, RED)],
                        denf_hbm.at[pl.ds(s * RED, RED)])

    pltpu.sync_copy(src_hbm.at[pl.ds(base, EPT)], src_v)
    pltpu.sync_copy(dst_hbm.at[pl.ds(base, EPT)], dst_v)
    pltpu.sync_copy(ex_hbm.at[pl.ds(base, EPT)], ex_v)

    def chunk(ci, _):
        slot = lax.rem(ci, 2)
        oslot = lax.rem(ci + 1, 2)
        cbase = ci * K

        def grp(g, _):
            off = cbase + g * L
            s16 = src_v[pl.ds(off, L)]
            d16 = dst_v[pl.ds(off, L)]
            e16 = ex_v[pl.ds(off, L)]
            dn = plsc.load_gather(den_v, [d16])
            alpha_full[pl.ds(off, L)] = e16 / dn
            fidx2_v[slot, pl.ds(g * L, L)] = s16 * jnp.int32(N) + d16
            return 0
        lax.fori_loop(0, K // L, grp, 0)

        @pl.when(ci >= 1)
        def _():
            pltpu.make_async_copy(alpha_full.at[pl.ds(0, K)],
                                  attn_hbm.at[fidx2_v.at[oslot]],
                                  asem).wait()

        pltpu.async_copy(alpha_full.at[pl.ds(cbase, K)],
                         attn_hbm.at[fidx2_v.at[slot]], asem)
        return 0
    lax.fori_loop(0, NCH, chunk, 0)

    pltpu.make_async_copy(alpha_full.at[pl.ds(0, K)],
                          attn_hbm.at[fidx2_v.at[0]], asem).wait()

    toff = NCH * K
    s16 = src_v[pl.ds(toff, L)]
    d16 = dst_v[pl.ds(toff, L)]
    e16 = ex_v[pl.ds(toff, L)]
    dn = plsc.load_gather(den_v, [d16])
    alpha_full[pl.ds(toff, L)] = e16 / dn
    f16 = s16 * jnp.int32(N) + d16
    pltpu.sync_copy(alpha_full.at[pl.ds(toff, L)], attn_hbm.at[f16])

    pltpu.sync_copy(alpha_full, alpha_hbm.at[pl.ds(base, EPT)])


_sc2a = _mpmd._mpmd_map(
    [(plsc.VectorSubcoreMesh(core_axis_name="c", subcore_axis_name="s"),
      _sc2a_body)],
    [
        jax.ShapeDtypeStruct((N * N,), jnp.float32),
        jax.ShapeDtypeStruct((E,), jnp.float32),
        jax.ShapeDtypeStruct((NPAD,), jnp.float32),
    ],
    input_output_aliases={5: 0},
    compiler_params=pltpu.CompilerParams(needs_layout_passes=False,
                                         use_tc_tiling_on_sc=False),
    scratch_types=[
        pltpu.VMEM((NPAD,), jnp.float32),
        pltpu.VMEM((NPAD,), jnp.float32),
        pltpu.VMEM((EPT,), jnp.int32),
        pltpu.VMEM((EPT,), jnp.int32),
        pltpu.VMEM((EPT,), jnp.float32),
        pltpu.VMEM((EPT,), jnp.float32),
        pltpu.VMEM((2, K), jnp.int32),
        pltpu.SemaphoreType.DMA,
    ],
)


def _sc3_body(src_hbm, dst_hbm, alpha_hbm, h2_hbm,
              agga0_hbm, agga1_hbm, aggb0_hbm, aggb1_hbm,
              src_v, dst_v, alpha_full,
              sidx_v, didx_v, alpha_v, rows_v, zrow_v,
              gsem, msem, agg_sh):
    c = lax.axis_index("c")
    s = lax.axis_index("s")
    wid = c * NS + s
    base = wid * EPT

    def zero_zrow(q, _):
        i = q // (DH // L)
        p = q % (DH // L)
        zrow_v[i, pl.ds(p * L, L)] = jnp.zeros((L,), jnp.float32)
        return 0
    lax.fori_loop(0, K * (DH // L), zero_zrow, 0)

    pltpu.sync_copy(src_hbm.at[pl.ds(base, EPT)], src_v)
    pltpu.sync_copy(dst_hbm.at[pl.ds(base, EPT)], dst_v)
    pltpu.sync_copy(alpha_hbm.at[pl.ds(base, EPT)], alpha_full)

    agg_outs = ((agga0_hbm, agga1_hbm), (aggb0_hbm, aggb1_hbm))

    for f in range(2):
        for i in range(RED // K):
            pltpu.sync_copy(zrow_v, agg_sh.at[pl.ds(s * RED + i * K, K)])
        plsc.subcore_barrier()

        def grp_fill(ci, slot):
            cbase = ci * K

            def grp(g, _):
                off = cbase + g * L
                o = g * L
                s16 = src_v[pl.ds(off, L)]
                alpha_v[slot, pl.ds(o, L)] = alpha_full[pl.ds(off, L)]
                sidx_v[slot, pl.ds(o, L)] = s16 * jnp.int32(2) + jnp.int32(f)
                didx_v[slot, pl.ds(o, L)] = dst_v[pl.ds(off, L)]
                return 0
            lax.fori_loop(0, K // L, grp, 0)

        def start_gather(ci, slot):
            grp_fill(ci, slot)
            pltpu.async_copy(h2_hbm.at[sidx_v.at[slot]], rows_v.at[slot],
                             gsem)

        start_gather(jnp.int32(0), jnp.int32(0))

        def chunk(ci, _):
            slot = lax.rem(ci, 2)
            oslot = lax.rem(ci + 1, 2)
            pltpu.make_async_copy(h2_hbm.at[sidx_v.at[slot]],
                                  rows_v.at[slot], gsem).wait()

            @pl.when(ci >= 1)
            def _():
                pltpu.make_async_copy(rows_v.at[oslot],
                                      agg_sh.at[didx_v.at[oslot]], msem).wait()

            @pl.when(ci + 1 < NCH)
            def _():
                start_gather(ci + 1, oslot)

            def scale(j, _):
                av = plsc.load_gather(
                    alpha_v, [jnp.full((L,), slot, jnp.int32),
                              jnp.full((L,), j, jnp.int32)])
                for p in range(DH // L):
                    r = rows_v[slot, j, pl.ds(p * L, L)]
                    rows_v[slot, j, pl.ds(p * L, L)] = r * av
                return 0
            lax.fori_loop(0, K, scale, 0)

            pltpu.async_copy(rows_v.at[slot], agg_sh.at[didx_v.at[slot]],
                             msem, add=True)
            return 0
        lax.fori_loop(0, NCH, chunk, 0)

        pltpu.make_async_copy(rows_v.at[lax.rem(NCH - 1, 2)],
                              agg_sh.at[didx_v.at[lax.rem(NCH - 1, 2)]],
                              msem).wait()

        toff = NCH * K
        s16 = src_v[pl.ds(toff, L)]
        d16 = dst_v[pl.ds(toff, L)]
        alpha_v[0, pl.ds(0, L)] = alpha_full[pl.ds(toff, L)]
        pltpu.sync_copy(h2_hbm.at[s16 * jnp.int32(2) + jnp.int32(f)],
                        rows_v.at[0, pl.ds(0, L)])

        def tscale(j, _):
            av = plsc.load_gather(
                alpha_v, [jnp.zeros((L,), jnp.int32),
                          jnp.full((L,), j, jnp.int32)])
            for p in range(DH // L):
                r = rows_v[0, j, pl.ds(p * L, L)]
                rows_v[0, j, pl.ds(p * L, L)] = r * av
            return 0
        lax.fori_loop(0, L, tscale, 0)

        pltpu.sync_copy(rows_v.at[0, pl.ds(0, L)], agg_sh.at[d16], add=True)

        plsc.subcore_barrier()

        @pl.when(c == 0)
        def _():
            pltpu.sync_copy(agg_sh.at[pl.ds(s * RED, RED)],
                            agg_outs[0][f].at[pl.ds(s * RED, RED)])

        @pl.when(c == 1)
        def _():
            pltpu.sync_copy(agg_sh.at[pl.ds(s * RED, RED)],
                            agg_outs[1][f].at[pl.ds(s * RED, RED)])

        plsc.subcore_barrier()


_sc3 = pl.kernel(
    _sc3_body,
    out_type=[
        jax.ShapeDtypeStruct((NPAD, DH), jnp.float32),
        jax.ShapeDtypeStruct((NPAD, DH), jnp.float32),
        jax.ShapeDtypeStruct((NPAD, DH), jnp.float32),
        jax.ShapeDtypeStruct((NPAD, DH), jnp.float32),
    ],
    mesh=plsc.VectorSubcoreMesh(core_axis_name="c", subcore_axis_name="s"),
    compiler_params=pltpu.CompilerParams(needs_layout_passes=False,
                                         use_tc_tiling_on_sc=False),
    scratch_types=[
        pltpu.VMEM((EPT,), jnp.int32),
        pltpu.VMEM((EPT,), jnp.int32),
        pltpu.VMEM((EPT,), jnp.float32),
        pltpu.VMEM((2, K), jnp.int32),
        pltpu.VMEM((2, K), jnp.int32),
        pltpu.VMEM((2, K), jnp.float32),
        pltpu.VMEM((2, K, DH), jnp.float32),
        pltpu.VMEM((K, DH), jnp.float32),
        pltpu.SemaphoreType.DMA,
        pltpu.SemaphoreType.DMA,
        pltpu.MemorySpace.VMEM_SHARED((NPAD, DH), jnp.float32),
    ],
)


def _tc2_body(a00_ref, a01_ref, b00_ref, b01_ref, den_ref, h_ref, o_ref):
    lo = a00_ref[...] + b00_ref[...]
    hi = a01_ref[...] + b01_ref[...]
    ssum = jnp.concatenate([lo, hi], axis=1)
    den = den_ref[...]
    o_ref[...] = jnp.where(den > 0, ssum, h_ref[...])


_tc2 = pl.pallas_call(
    _tc2_body,
    grid=(pl.cdiv(N, BN),),
    in_specs=[
        pl.BlockSpec((BN, DH), lambda i: (i, 0)),
        pl.BlockSpec((BN, DH), lambda i: (i, 0)),
        pl.BlockSpec((BN, DH), lambda i: (i, 0)),
        pl.BlockSpec((BN, DH), lambda i: (i, 0)),
        pl.BlockSpec((BN, 1), lambda i: (i, 0)),
        pl.BlockSpec((BN, D), lambda i: (i, 0)),
    ],
    out_specs=pl.BlockSpec((BN, D), lambda i: (i, 0)),
    out_shape=jax.ShapeDtypeStruct((N, D), jnp.float32),
)


def kernel(x, edge_index, W, att_src, att_dst):
    a8 = jnp.zeros((8, D), jnp.float32).at[0].set(att_src[0]).at[1].set(att_dst[0])
    h, asadT = _tc1(x, W, a8)
    src = edge_index[0]
    dst = edge_index[1]
    as_arr = asadT[0]
    ad_arr = asadT[1]
    ex, den_a, den_b = _sc1(src, dst, as_arr, ad_arr)
    zer = jnp.zeros((N * N,), jnp.float32)
    attnflat, alpha_e, denf = _sc2a(src, dst, ex, den_a, den_b, zer)
    h2 = jnp.reshape(h, (2 * N, DH))
    agg_a0, agg_a1, agg_b0, agg_b1 = _sc3(src, dst, alpha_e, h2)
    out = _tc2(agg_a0, agg_a1, agg_b0, agg_b1,
               jnp.reshape(denf[:N], (N, 1)), h)
    return out, attnflat.reshape(N, N)
